# baseline (speedup 1.0000x reference)
import jax
import jax.numpy as jnp
from jax.experimental import pallas as pl
from jax.experimental.pallas import tpu as pltpu

_RB = 256


def _gcn_body(adj_ref, feat_ref, w1_ref, b1_ref, w2_ref, b2_ref, out_ref,
              g_ref, h_ref, c_ref):
    r = pl.program_id(1)
    nr = pl.num_programs(1)
    adj = adj_ref[0]
    deg = jnp.maximum(jnp.sum(adj, axis=1, keepdims=True), 1.0)
    invdeg = 1.0 / deg

    @pl.when(r == 0)
    def _init():
        g_ref[...] = jnp.dot(feat_ref[0], w1_ref[...],
                             preferred_element_type=jnp.float32)

    m = jnp.dot(adj, g_ref[...], preferred_element_type=jnp.float32)
    h_ref[pl.ds(r * _RB, _RB), :] = jnp.maximum(m * invdeg + b1_ref[...], 0.0)

    c_blk = jax.lax.dot_general(invdeg, adj, (((0,), (0,)), ((), ())),
                                preferred_element_type=jnp.float32)

    @pl.when(r == 0)
    def _cinit():
        c_ref[...] = c_blk

    @pl.when(r > 0)
    def _cacc():
        c_ref[...] += c_blk

    @pl.when(r == nr - 1)
    def _finish():
        s = h_ref.shape[0]
        y = jnp.dot(c_ref[...], h_ref[...],
                    preferred_element_type=jnp.float32) * (1.0 / s)
        y = jnp.dot(y, w2_ref[...],
                    preferred_element_type=jnp.float32) + b2_ref[...]
        nrm = jnp.sqrt(jnp.sum(y * y))
        out_ref[0] = y / jnp.maximum(nrm, 1e-5)


@jax.jit
def kernel(adj, n_feat, W1, b1, W2, b2):
    B, S, _ = adj.shape
    FT = n_feat.shape[-1]
    H = W1.shape[-1]
    O = W2.shape[-1]
    R = S // _RB
    b1r = b1.reshape(1, H)
    b2r = b2.reshape(1, O)
    return pl.pallas_call(
        _gcn_body,
        grid=(B, R),
        in_specs=[
            pl.BlockSpec((1, _RB, S), lambda b, r: (b, r, 0)),
            pl.BlockSpec((1, S, FT), lambda b, r: (b, 0, 0)),
            pl.BlockSpec((FT, H), lambda b, r: (0, 0)),
            pl.BlockSpec((1, H), lambda b, r: (0, 0)),
            pl.BlockSpec((H, O), lambda b, r: (0, 0)),
            pl.BlockSpec((1, O), lambda b, r: (0, 0)),
        ],
        out_specs=pl.BlockSpec((1, 1, O), lambda b, r: (b, 0, 0)),
        out_shape=jax.ShapeDtypeStruct((B, 1, O), jnp.float32),
        scratch_shapes=[
            pltpu.VMEM((S, H), jnp.float32),
            pltpu.VMEM((S, H), jnp.float32),
            pltpu.VMEM((1, S), jnp.float32),
        ],
        compiler_params=pltpu.CompilerParams(
            dimension_semantics=("parallel", "arbitrary")),
    )(adj, n_feat, W1, b1r, W2, b2r).reshape(B, O)


# speedup vs baseline: 2.2496x; 2.2496x over previous
import jax
import jax.numpy as jnp
from jax.experimental import pallas as pl
from jax.experimental.pallas import tpu as pltpu

_BPB = 2


def _gcn_body(adj_ref, feat_ref, w1_ref, b1_ref, w2_ref, b2_ref, out_ref):
    adj = adj_ref[...]
    feat = feat_ref[...]
    s = adj.shape[-1]
    deg = jnp.maximum(jnp.sum(adj, axis=2, keepdims=True), 1.0)
    invdeg = 1.0 / deg
    g = jax.lax.dot_general(feat, w1_ref[...], (((2,), (0,)), ((), ())),
                            preferred_element_type=jnp.float32)
    m = jax.lax.dot_general(adj, g, (((2,), (1,)), ((0,), (0,))),
                            preferred_element_type=jnp.float32)
    h = jnp.maximum(m * invdeg + b1_ref[...], 0.0)
    c = jax.lax.dot_general(invdeg, adj, (((1,), (1,)), ((0,), (0,))),
                            preferred_element_type=jnp.float32)
    y = jax.lax.dot_general(c, h, (((2,), (1,)), ((0,), (0,))),
                            preferred_element_type=jnp.float32) * (1.0 / s)
    y = jax.lax.dot_general(y, w2_ref[...], (((2,), (0,)), ((), ())),
                            preferred_element_type=jnp.float32) + b2_ref[...]
    nrm = jnp.sqrt(jnp.sum(y * y, axis=-1, keepdims=True))
    out_ref[...] = y / jnp.maximum(nrm, 1e-5)


@jax.jit
def kernel(adj, n_feat, W1, b1, W2, b2):
    B, S, _ = adj.shape
    FT = n_feat.shape[-1]
    H = W1.shape[-1]
    O = W2.shape[-1]
    b1r = b1.reshape(1, H)
    b2r = b2.reshape(1, O)
    return pl.pallas_call(
        _gcn_body,
        grid=(B // _BPB,),
        in_specs=[
            pl.BlockSpec((_BPB, S, S), lambda b: (b, 0, 0)),
            pl.BlockSpec((_BPB, S, FT), lambda b: (b, 0, 0)),
            pl.BlockSpec((FT, H), lambda b: (0, 0)),
            pl.BlockSpec((1, H), lambda b: (0, 0)),
            pl.BlockSpec((H, O), lambda b: (0, 0)),
            pl.BlockSpec((1, O), lambda b: (0, 0)),
        ],
        out_specs=pl.BlockSpec((_BPB, 1, O), lambda b: (b, 0, 0)),
        out_shape=jax.ShapeDtypeStruct((B, 1, O), jnp.float32),
        compiler_params=pltpu.CompilerParams(
            dimension_semantics=("parallel",)),
    )(adj, n_feat, W1, b1r, W2, b2r).reshape(B, O)


# speedup vs baseline: 2.3166x; 1.0298x over previous
import jax
import jax.numpy as jnp
from jax.experimental import pallas as pl
from jax.experimental.pallas import tpu as pltpu

_BPB = 4


def _gcn_body(adj_ref, feat_ref, w1_ref, b1_ref, w2_ref, b2_ref, out_ref):
    adj = adj_ref[...]
    feat = feat_ref[...]
    s = adj.shape[-1]
    deg = jnp.maximum(jnp.sum(adj, axis=2, keepdims=True), 1.0)
    invdeg = 1.0 / deg
    g = jax.lax.dot_general(feat, w1_ref[...], (((2,), (0,)), ((), ())),
                            preferred_element_type=jnp.float32)
    m = jax.lax.dot_general(adj, g, (((2,), (1,)), ((0,), (0,))),
                            preferred_element_type=jnp.float32)
    h = jnp.maximum(m * invdeg + b1_ref[...], 0.0)
    c = jax.lax.dot_general(invdeg, adj, (((1,), (1,)), ((0,), (0,))),
                            preferred_element_type=jnp.float32)
    y = jax.lax.dot_general(c, h, (((2,), (1,)), ((0,), (0,))),
                            preferred_element_type=jnp.float32) * (1.0 / s)
    y = jax.lax.dot_general(y, w2_ref[...], (((2,), (0,)), ((), ())),
                            preferred_element_type=jnp.float32) + b2_ref[...]
    nrm = jnp.sqrt(jnp.sum(y * y, axis=-1, keepdims=True))
    out_ref[...] = y / jnp.maximum(nrm, 1e-5)


@jax.jit
def kernel(adj, n_feat, W1, b1, W2, b2):
    B, S, _ = adj.shape
    FT = n_feat.shape[-1]
    H = W1.shape[-1]
    O = W2.shape[-1]
    b1r = b1.reshape(1, H)
    b2r = b2.reshape(1, O)
    return pl.pallas_call(
        _gcn_body,
        grid=(B // _BPB,),
        in_specs=[
            pl.BlockSpec((_BPB, S, S), lambda b: (b, 0, 0)),
            pl.BlockSpec((_BPB, S, FT), lambda b: (b, 0, 0)),
            pl.BlockSpec((FT, H), lambda b: (0, 0)),
            pl.BlockSpec((1, H), lambda b: (0, 0)),
            pl.BlockSpec((H, O), lambda b: (0, 0)),
            pl.BlockSpec((1, O), lambda b: (0, 0)),
        ],
        out_specs=pl.BlockSpec((_BPB, 1, O), lambda b: (b, 0, 0)),
        out_shape=jax.ShapeDtypeStruct((B, 1, O), jnp.float32),
        compiler_params=pltpu.CompilerParams(
            dimension_semantics=("parallel",)),
    )(adj, n_feat, W1, b1r, W2, b2r).reshape(B, O)
